# Initial kernel scaffold; baseline (speedup 1.0000x reference)
import jax
import jax.numpy as jnp
from jax.experimental import pallas as pl


def kernel(x, p):
    raise NotImplementedError("write your pallas kernel here")


# speedup vs baseline: 1.7389x; 1.7389x over previous
import functools

import jax
import jax.numpy as jnp
from jax import lax
from jax.experimental import pallas as pl
from jax.experimental.pallas import tpu as pltpu
from jax.experimental.pallas import tpu_sc as plsc

N = 1000000
ROWS = N // 128
PAD_N = 1048576
PAD_ROWS = PAD_N // 128
CHUNK_ROWS = 512
N_CHUNKS = PAD_ROWS // CHUNK_ROWS

TOT_X = 16384 * 100
NW = 32
PER_W = TOT_X // NW
CH = 2048
N_ITERS = PER_W // CH


def _max_kernel(p_ref, m_ref):
    i = pl.program_id(0)

    @pl.when(i == 0)
    def _():
        m_ref[0, 0] = jnp.float32(-jnp.inf)

    m_ref[0, 0] = jnp.maximum(m_ref[0, 0], jnp.max(p_ref[...]))


def _table_kernel(m_ref, p_ref, steps_ref, delta_ref, tot_ref, carry_ref):
    i = pl.program_id(0)

    @pl.when(i == 0)
    def _():
        carry_ref[0, 0] = jnp.float32(0.0)

    e = jnp.exp(p_ref[...] - m_ref[0, 0])

    ids = lax.broadcasted_iota(jnp.int32, (128, 128), 0)
    jds = lax.broadcasted_iota(jnp.int32, (128, 128), 1)
    U = (ids <= jds).astype(jnp.float32)

    C = jnp.dot(e, U, preferred_element_type=jnp.float32)
    r_col = C[:, 127:128]

    ids2 = lax.broadcasted_iota(jnp.int32, (CHUNK_ROWS, CHUNK_ROWS), 0)
    jds2 = lax.broadcasted_iota(jnp.int32, (CHUNK_ROWS, CHUNK_ROWS), 1)
    Ls = (jds2 < ids2).astype(jnp.float32)
    offs = jnp.dot(Ls, r_col, preferred_element_type=jnp.float32)

    carry = carry_ref[0, 0]
    ce = C + offs + carry
    steps_ref[...] = ce
    delta_ref[...] = e
    new_carry = carry + jnp.sum(r_col)
    carry_ref[0, 0] = new_carry
    tot_ref[0, 0] = new_carry


def _build_table(p):
    p_pad = jnp.full((PAD_N,), -jnp.inf, dtype=jnp.float32).at[:N].set(p)
    p2 = p_pad.reshape(PAD_ROWS, 128)

    m = pl.pallas_call(
        _max_kernel,
        grid=(N_CHUNKS,),
        in_specs=[pl.BlockSpec((CHUNK_ROWS, 128), lambda i: (i, 0))],
        out_specs=pl.BlockSpec(memory_space=pltpu.SMEM),
        out_shape=jax.ShapeDtypeStruct((1, 1), jnp.float32),
    )(p2)

    steps, delta, tot = pl.pallas_call(
        _table_kernel,
        grid=(N_CHUNKS,),
        in_specs=[
            pl.BlockSpec(memory_space=pltpu.SMEM),
            pl.BlockSpec((CHUNK_ROWS, 128), lambda i: (i, 0)),
        ],
        out_specs=[
            pl.BlockSpec((CHUNK_ROWS, 128), lambda i: (i, 0)),
            pl.BlockSpec((CHUNK_ROWS, 128), lambda i: (i, 0)),
            pl.BlockSpec(memory_space=pltpu.SMEM),
        ],
        out_shape=[
            jax.ShapeDtypeStruct((PAD_ROWS, 128), jnp.float32),
            jax.ShapeDtypeStruct((PAD_ROWS, 128), jnp.float32),
            jax.ShapeDtypeStruct((1, 1), jnp.float32),
        ],
        scratch_shapes=[pltpu.SMEM((1, 1), jnp.float32)],
    )(m, p2)
    return steps.reshape(PAD_N), delta.reshape(PAD_N), tot


def _gather_body(xf, steps, delta, tot, out, xv, idxv, fracv, sv, dv, wv, totv,
                 sem_s, sem_d):
    wid = lax.axis_index("s") * 2 + lax.axis_index("c")
    pltpu.sync_copy(tot, totv)
    inv = 1.0 / totv[...]

    def chunk_body(c, _):
        base = wid * PER_W + c * CH
        pltpu.sync_copy(xf.at[pl.ds(base, CH)], xv)

        def idx_body(j, _):
            xs = xv[pl.ds(j * 16, 16)]
            t = xs * jnp.float32(N) - jnp.float32(1e-5)
            n = t.astype(jnp.int32)
            idxv[pl.ds(j * 16, 16)] = n
            fracv[pl.ds(j * 16, 16)] = (
                xs * jnp.float32(N) - n.astype(jnp.float32) - 1.0)
            return 0

        lax.fori_loop(0, CH // 16, idx_body, 0)

        cp_s = pltpu.async_copy(steps.at[idxv], sv, sem_s)
        cp_d = pltpu.async_copy(delta.at[idxv], dv, sem_d)
        cp_s.wait()
        cp_d.wait()

        def w_body(j, _):
            s = sv[pl.ds(j * 16, 16)]
            d = dv[pl.ds(j * 16, 16)]
            f = fracv[pl.ds(j * 16, 16)]
            wv[pl.ds(j * 16, 16)] = (s + f * d) * inv
            return 0

        lax.fori_loop(0, CH // 16, w_body, 0)
        pltpu.sync_copy(wv, out.at[pl.ds(base, CH)])
        return 0

    lax.fori_loop(0, N_ITERS, chunk_body, 0)


@functools.lru_cache(maxsize=None)
def _make_gather_kernel():
    @functools.partial(
        pl.kernel,
        out_type=jax.ShapeDtypeStruct((TOT_X,), jnp.float32),
        mesh=plsc.VectorSubcoreMesh(core_axis_name="c", subcore_axis_name="s"),
        scratch_types=[
            pltpu.VMEM((CH,), jnp.float32),
            pltpu.VMEM((CH,), jnp.int32),
            pltpu.VMEM((CH,), jnp.float32),
            pltpu.VMEM((CH,), jnp.float32),
            pltpu.VMEM((CH,), jnp.float32),
            pltpu.VMEM((CH,), jnp.float32),
            pltpu.VMEM((16,), jnp.float32),
            pltpu.SemaphoreType.DMA,
            pltpu.SemaphoreType.DMA,
        ],
    )
    def _gather_kernel(xf, steps, delta, tot, out, *scratch):
        _gather_body(xf, steps, delta, tot, out, *scratch)

    return _gather_kernel


def kernel(x, p):
    steps, delta, tot = _build_table(p)
    tot16 = jnp.broadcast_to(tot.reshape(1), (16,))
    xf = x.reshape(TOT_X)
    w = _make_gather_kernel()(xf, steps, delta, tot16)
    return w.reshape(x.shape)


# speedup vs baseline: 1.8204x; 1.0468x over previous
import functools

import jax
import jax.numpy as jnp
from jax import lax
from jax.experimental import pallas as pl
from jax.experimental.pallas import tpu as pltpu
from jax.experimental.pallas import tpu_sc as plsc

N = 1000000
ROWS = N // 128
PAD_N = 1048576
PAD_ROWS = PAD_N // 128
CHUNK_ROWS = 512
N_CHUNKS = PAD_ROWS // CHUNK_ROWS

TOT_X = 16384 * 100
NW = 32
PER_W = TOT_X // NW
CH = 2048
N_ITERS = PER_W // CH


def _max_kernel(p_ref, m_ref):
    i = pl.program_id(0)

    @pl.when(i == 0)
    def _():
        m_ref[0, 0] = jnp.float32(-jnp.inf)

    m_ref[0, 0] = jnp.maximum(m_ref[0, 0], jnp.max(p_ref[...]))


def _table_kernel(m_ref, p_ref, steps_ref, delta_ref, tot_ref, carry_ref):
    i = pl.program_id(0)

    @pl.when(i == 0)
    def _():
        carry_ref[0, 0] = jnp.float32(0.0)

    e = jnp.exp(p_ref[...] - m_ref[0, 0])

    ids = lax.broadcasted_iota(jnp.int32, (128, 128), 0)
    jds = lax.broadcasted_iota(jnp.int32, (128, 128), 1)
    U = (ids <= jds).astype(jnp.float32)

    C = jnp.dot(e, U, preferred_element_type=jnp.float32)
    r_col = C[:, 127:128]

    ids2 = lax.broadcasted_iota(jnp.int32, (CHUNK_ROWS, CHUNK_ROWS), 0)
    jds2 = lax.broadcasted_iota(jnp.int32, (CHUNK_ROWS, CHUNK_ROWS), 1)
    Ls = (jds2 < ids2).astype(jnp.float32)
    offs = jnp.dot(Ls, r_col, preferred_element_type=jnp.float32)

    carry = carry_ref[0, 0]
    ce = C + offs + carry
    steps_ref[...] = ce
    delta_ref[...] = e
    new_carry = carry + jnp.sum(r_col)
    carry_ref[0, 0] = new_carry
    tot_ref[0, 0] = new_carry


def _build_table(p):
    p_pad = jnp.full((PAD_N,), -jnp.inf, dtype=jnp.float32).at[:N].set(p)
    p2 = p_pad.reshape(PAD_ROWS, 128)

    m = pl.pallas_call(
        _max_kernel,
        grid=(N_CHUNKS,),
        in_specs=[pl.BlockSpec((CHUNK_ROWS, 128), lambda i: (i, 0))],
        out_specs=pl.BlockSpec(memory_space=pltpu.SMEM),
        out_shape=jax.ShapeDtypeStruct((1, 1), jnp.float32),
    )(p2)

    steps, delta, tot = pl.pallas_call(
        _table_kernel,
        grid=(N_CHUNKS,),
        in_specs=[
            pl.BlockSpec(memory_space=pltpu.SMEM),
            pl.BlockSpec((CHUNK_ROWS, 128), lambda i: (i, 0)),
        ],
        out_specs=[
            pl.BlockSpec((CHUNK_ROWS, 128), lambda i: (i, 0)),
            pl.BlockSpec((CHUNK_ROWS, 128), lambda i: (i, 0)),
            pl.BlockSpec(memory_space=pltpu.SMEM),
        ],
        out_shape=[
            jax.ShapeDtypeStruct((PAD_ROWS, 128), jnp.float32),
            jax.ShapeDtypeStruct((PAD_ROWS, 128), jnp.float32),
            jax.ShapeDtypeStruct((1, 1), jnp.float32),
        ],
        scratch_shapes=[pltpu.SMEM((1, 1), jnp.float32)],
    )(m, p2)
    return steps.reshape(PAD_N), delta.reshape(PAD_N), tot


def _gather_body(xf, steps, delta, tot, out, xv, idxv, fracv, sv, dv, wv, totv,
                 sem_s, sem_d):
    wid = lax.axis_index("s") * 2 + lax.axis_index("c")
    pltpu.sync_copy(tot, totv)
    inv = 1.0 / totv[...]

    def chunk_body(c, _):
        base = wid * PER_W + c * CH
        pltpu.sync_copy(xf.at[pl.ds(base, CH)], xv)

        @plsc.parallel_loop(0, CH, 16, unroll=8)
        def idx_body(j):
            xs = xv[pl.ds(j, 16)]
            t = xs * jnp.float32(N) - jnp.float32(1e-5)
            n = t.astype(jnp.int32)
            idxv[pl.ds(j, 16)] = n
            fracv[pl.ds(j, 16)] = (
                xs * jnp.float32(N) - n.astype(jnp.float32) - 1.0)

        cp_s = pltpu.async_copy(steps.at[idxv], sv, sem_s)
        cp_d = pltpu.async_copy(delta.at[idxv], dv, sem_d)
        cp_s.wait()
        cp_d.wait()

        @plsc.parallel_loop(0, CH, 16, unroll=8)
        def w_body(j):
            s = sv[pl.ds(j, 16)]
            d = dv[pl.ds(j, 16)]
            f = fracv[pl.ds(j, 16)]
            wv[pl.ds(j, 16)] = (s + f * d) * inv
        pltpu.sync_copy(wv, out.at[pl.ds(base, CH)])
        return 0

    lax.fori_loop(0, N_ITERS, chunk_body, 0)


@functools.lru_cache(maxsize=None)
def _make_gather_kernel():
    @functools.partial(
        pl.kernel,
        out_type=jax.ShapeDtypeStruct((TOT_X,), jnp.float32),
        mesh=plsc.VectorSubcoreMesh(core_axis_name="c", subcore_axis_name="s"),
        scratch_types=[
            pltpu.VMEM((CH,), jnp.float32),
            pltpu.VMEM((CH,), jnp.int32),
            pltpu.VMEM((CH,), jnp.float32),
            pltpu.VMEM((CH,), jnp.float32),
            pltpu.VMEM((CH,), jnp.float32),
            pltpu.VMEM((CH,), jnp.float32),
            pltpu.VMEM((16,), jnp.float32),
            pltpu.SemaphoreType.DMA,
            pltpu.SemaphoreType.DMA,
        ],
    )
    def _gather_kernel(xf, steps, delta, tot, out, *scratch):
        _gather_body(xf, steps, delta, tot, out, *scratch)

    return _gather_kernel


def kernel(x, p):
    steps, delta, tot = _build_table(p)
    tot16 = jnp.broadcast_to(tot.reshape(1), (16,))
    xf = x.reshape(TOT_X)
    w = _make_gather_kernel()(xf, steps, delta, tot16)
    return w.reshape(x.shape)


# speedup vs baseline: 2.1211x; 1.1652x over previous
import functools

import jax
import jax.numpy as jnp
from jax import lax
from jax.experimental import pallas as pl
from jax.experimental.pallas import tpu as pltpu
from jax.experimental.pallas import tpu_sc as plsc

N = 1000000
ROWS = N // 128
PAD_N = 1048576
PAD_ROWS = PAD_N // 128
CHUNK_ROWS = 512
N_CHUNKS = PAD_ROWS // CHUNK_ROWS

TOT_X = 16384 * 100
NW = 32
PER_W = TOT_X // NW
CH = 2048
N_ITERS = PER_W // CH


def _max_kernel(p_ref, m_ref):
    i = pl.program_id(0)

    @pl.when(i == 0)
    def _():
        m_ref[0, 0] = jnp.float32(-jnp.inf)

    m_ref[0, 0] = jnp.maximum(m_ref[0, 0], jnp.max(p_ref[...]))


def _table_kernel(m_ref, p_ref, pair_ref, tot_ref, carry_ref):
    i = pl.program_id(0)

    @pl.when(i == 0)
    def _():
        carry_ref[0, 0] = jnp.float32(0.0)

    e = jnp.exp(p_ref[...] - m_ref[0, 0])

    ids = lax.broadcasted_iota(jnp.int32, (128, 128), 0)
    jds = lax.broadcasted_iota(jnp.int32, (128, 128), 1)
    U = (ids <= jds).astype(jnp.float32)

    C = jnp.dot(e, U, preferred_element_type=jnp.float32)
    r_col = C[:, 127:128]

    ids2 = lax.broadcasted_iota(jnp.int32, (CHUNK_ROWS, CHUNK_ROWS), 0)
    jds2 = lax.broadcasted_iota(jnp.int32, (CHUNK_ROWS, CHUNK_ROWS), 1)
    Ls = (jds2 < ids2).astype(jnp.float32)
    offs = jnp.dot(Ls, r_col, preferred_element_type=jnp.float32)

    carry = carry_ref[0, 0]
    ce = C + offs + carry

    a_bits = lax.bitcast_convert_type(ce - e, jnp.int32)
    e_bits = lax.bitcast_convert_type(e, jnp.int32)
    au = (a_bits + 0x8000) >> 16
    eu = (e_bits + 0x8000) >> 16
    pair_ref[...] = (au << 16) | (eu & 0xFFFF)

    new_carry = carry + jnp.sum(r_col)
    carry_ref[0, 0] = new_carry
    tot_ref[0, 0] = new_carry


def _build_table(p):
    p_pad = jnp.full((PAD_N,), -jnp.inf, dtype=jnp.float32).at[:N].set(p)
    p2 = p_pad.reshape(PAD_ROWS, 128)

    m = pl.pallas_call(
        _max_kernel,
        grid=(N_CHUNKS,),
        in_specs=[pl.BlockSpec((CHUNK_ROWS, 128), lambda i: (i, 0))],
        out_specs=pl.BlockSpec(memory_space=pltpu.SMEM),
        out_shape=jax.ShapeDtypeStruct((1, 1), jnp.float32),
    )(p2)

    pairs, tot = pl.pallas_call(
        _table_kernel,
        grid=(N_CHUNKS,),
        in_specs=[
            pl.BlockSpec(memory_space=pltpu.SMEM),
            pl.BlockSpec((CHUNK_ROWS, 128), lambda i: (i, 0)),
        ],
        out_specs=[
            pl.BlockSpec((CHUNK_ROWS, 128), lambda i: (i, 0)),
            pl.BlockSpec(memory_space=pltpu.SMEM),
        ],
        out_shape=[
            jax.ShapeDtypeStruct((PAD_ROWS, 128), jnp.int32),
            jax.ShapeDtypeStruct((1, 1), jnp.float32),
        ],
        scratch_shapes=[pltpu.SMEM((1, 1), jnp.float32)],
    )(m, p2)
    return pairs.reshape(PAD_N), tot


def _gather_body(xf, pairs, tot, out, xv, idxv, fracv, pv, wv, totv, sem_p):
    wid = lax.axis_index("s") * 2 + lax.axis_index("c")
    pltpu.sync_copy(tot, totv)
    inv = 1.0 / totv[...]

    def chunk_body(c, _):
        base = wid * PER_W + c * CH
        pltpu.sync_copy(xf.at[pl.ds(base, CH)], xv)

        @plsc.parallel_loop(0, CH, 16, unroll=8)
        def idx_body(j):
            xs = xv[pl.ds(j, 16)]
            t = xs * jnp.float32(N) - jnp.float32(1e-5)
            n = t.astype(jnp.int32)
            idxv[pl.ds(j, 16)] = n
            fracv[pl.ds(j, 16)] = (
                xs * jnp.float32(N) - n.astype(jnp.float32))

        pltpu.async_copy(pairs.at[idxv], pv, sem_p).wait()

        @plsc.parallel_loop(0, CH, 16, unroll=8)
        def w_body(j):
            v = pv[pl.ds(j, 16)]
            a = plsc.bitcast(v & jnp.int32(-65536), jnp.float32)
            e = plsc.bitcast(v << 16, jnp.float32)
            g = fracv[pl.ds(j, 16)]
            wv[pl.ds(j, 16)] = (a + g * e) * inv

        pltpu.sync_copy(wv, out.at[pl.ds(base, CH)])
        return 0

    lax.fori_loop(0, N_ITERS, chunk_body, 0)


@functools.lru_cache(maxsize=None)
def _make_gather_kernel():
    @functools.partial(
        pl.kernel,
        out_type=jax.ShapeDtypeStruct((TOT_X,), jnp.float32),
        mesh=plsc.VectorSubcoreMesh(core_axis_name="c", subcore_axis_name="s"),
        compiler_params=pltpu.CompilerParams(needs_layout_passes=False),
        scratch_types=[
            pltpu.VMEM((CH,), jnp.float32),
            pltpu.VMEM((CH,), jnp.int32),
            pltpu.VMEM((CH,), jnp.float32),
            pltpu.VMEM((CH,), jnp.int32),
            pltpu.VMEM((CH,), jnp.float32),
            pltpu.VMEM((16,), jnp.float32),
            pltpu.SemaphoreType.DMA,
        ],
    )
    def _gather_kernel(xf, pairs, tot, out, *scratch):
        _gather_body(xf, pairs, tot, out, *scratch)

    return _gather_kernel


def kernel(x, p):
    pairs, tot = _build_table(p)
    tot16 = jnp.broadcast_to(tot.reshape(1), (16,))
    xf = x.reshape(TOT_X)
    w = _make_gather_kernel()(xf, pairs, tot16)
    return w.reshape(x.shape)


# speedup vs baseline: 2.5124x; 1.1845x over previous
import functools

import jax
import jax.numpy as jnp
from jax import lax
from jax.experimental import pallas as pl
from jax.experimental.pallas import tpu as pltpu
from jax.experimental.pallas import tpu_sc as plsc

N = 1000000
ROWS = N // 128
PAD_N = 1048576
PAD_ROWS = PAD_N // 128
CHUNK_ROWS = 512
N_CHUNKS = PAD_ROWS // CHUNK_ROWS

TOT_X = 16384 * 100
NW = 32
PER_W = TOT_X // NW
CH = 3200
N_ITERS = PER_W // CH


def _max_kernel(p_ref, m_ref):
    i = pl.program_id(0)

    @pl.when(i == 0)
    def _():
        m_ref[0, 0] = jnp.float32(-jnp.inf)

    m_ref[0, 0] = jnp.maximum(m_ref[0, 0], jnp.max(p_ref[...]))


def _table_kernel(m_ref, p_ref, pair_ref, tot_ref, carry_ref):
    i = pl.program_id(0)

    @pl.when(i == 0)
    def _():
        carry_ref[0, 0] = jnp.float32(0.0)

    e = jnp.exp(p_ref[...] - m_ref[0, 0])

    ids = lax.broadcasted_iota(jnp.int32, (128, 128), 0)
    jds = lax.broadcasted_iota(jnp.int32, (128, 128), 1)
    U = (ids <= jds).astype(jnp.float32)

    C = jnp.dot(e, U, preferred_element_type=jnp.float32)
    r_col = C[:, 127:128]

    ids2 = lax.broadcasted_iota(jnp.int32, (CHUNK_ROWS, CHUNK_ROWS), 0)
    jds2 = lax.broadcasted_iota(jnp.int32, (CHUNK_ROWS, CHUNK_ROWS), 1)
    Ls = (jds2 < ids2).astype(jnp.float32)
    offs = jnp.dot(Ls, r_col, preferred_element_type=jnp.float32)

    carry = carry_ref[0, 0]
    ce = C + offs + carry

    a_bits = lax.bitcast_convert_type(ce - e, jnp.int32)
    e_bits = lax.bitcast_convert_type(e, jnp.int32)
    au = (a_bits + 0x8000) >> 16
    eu = (e_bits + 0x8000) >> 16
    pair_ref[...] = (au << 16) | (eu & 0xFFFF)

    new_carry = carry + jnp.sum(r_col)
    carry_ref[0, 0] = new_carry
    tot_ref[0, 0] = new_carry


def _build_table(p):
    p_pad = jnp.full((PAD_N,), -jnp.inf, dtype=jnp.float32).at[:N].set(p)
    p2 = p_pad.reshape(PAD_ROWS, 128)

    m = pl.pallas_call(
        _max_kernel,
        grid=(N_CHUNKS,),
        in_specs=[pl.BlockSpec((CHUNK_ROWS, 128), lambda i: (i, 0))],
        out_specs=pl.BlockSpec(memory_space=pltpu.SMEM),
        out_shape=jax.ShapeDtypeStruct((1, 1), jnp.float32),
    )(p2)

    pairs, tot = pl.pallas_call(
        _table_kernel,
        grid=(N_CHUNKS,),
        in_specs=[
            pl.BlockSpec(memory_space=pltpu.SMEM),
            pl.BlockSpec((CHUNK_ROWS, 128), lambda i: (i, 0)),
        ],
        out_specs=[
            pl.BlockSpec((CHUNK_ROWS, 128), lambda i: (i, 0)),
            pl.BlockSpec(memory_space=pltpu.SMEM),
        ],
        out_shape=[
            jax.ShapeDtypeStruct((PAD_ROWS, 128), jnp.int32),
            jax.ShapeDtypeStruct((1, 1), jnp.float32),
        ],
        scratch_shapes=[pltpu.SMEM((1, 1), jnp.float32)],
    )(m, p2)
    return pairs.reshape(PAD_N), tot


def _gather_body(xf, pairs, tot, out, xv, idxv, fracv, pv, wv, totv,
                 sem_x, sem_g, sem_w):
    wid = lax.axis_index("s") * 2 + lax.axis_index("c")
    pltpu.sync_copy(tot, totv)
    inv = 1.0 / totv[...]

    def base(c):
        return wid * PER_W + c * CH

    def cidx(b):
        @plsc.parallel_loop(0, CH, 16, unroll=8)
        def idx_body(j):
            xs = xv[b][pl.ds(j, 16)]
            t = xs * jnp.float32(N) - jnp.float32(1e-5)
            n = t.astype(jnp.int32)
            idxv[b][pl.ds(j, 16)] = n
            fracv[b][pl.ds(j, 16)] = (
                xs * jnp.float32(N) - n.astype(jnp.float32))

    def cw(b):
        @plsc.parallel_loop(0, CH, 16, unroll=8)
        def w_body(j):
            v = pv[b][pl.ds(j, 16)]
            a = plsc.bitcast(v & jnp.int32(-65536), jnp.float32)
            e = plsc.bitcast(v << 16, jnp.float32)
            g = fracv[b][pl.ds(j, 16)]
            wv[b][pl.ds(j, 16)] = (a + g * e) * inv

    x_cp = [None, None]
    g_cp = [None, None]
    w_cp = [None, None]

    x_cp[0] = pltpu.async_copy(xf.at[pl.ds(base(0), CH)], xv[0], sem_x[0])
    for c in range(N_ITERS):
        b = c & 1
        nb = b ^ 1
        x_cp[b].wait()
        cidx(b)
        if c >= 1:
            g_cp[nb].wait()
        g_cp[b] = pltpu.async_copy(pairs.at[idxv[b]], pv[b], sem_g[b])
        if c + 1 < N_ITERS:
            x_cp[nb] = pltpu.async_copy(
                xf.at[pl.ds(base(c + 1), CH)], xv[nb], sem_x[nb])
        if c >= 1:
            if w_cp[nb] is not None:
                w_cp[nb].wait()
            cw(nb)
            w_cp[nb] = pltpu.async_copy(
                wv[nb], out.at[pl.ds(base(c - 1), CH)], sem_w[nb])
    lb = (N_ITERS - 1) & 1
    g_cp[lb].wait()
    if w_cp[lb] is not None:
        w_cp[lb].wait()
    cw(lb)
    w_cp[lb] = pltpu.async_copy(
        wv[lb], out.at[pl.ds(base(N_ITERS - 1), CH)], sem_w[lb])
    w_cp[lb ^ 1].wait()
    w_cp[lb].wait()


@functools.lru_cache(maxsize=None)
def _make_gather_kernel():
    @functools.partial(
        pl.kernel,
        out_type=jax.ShapeDtypeStruct((TOT_X,), jnp.float32),
        mesh=plsc.VectorSubcoreMesh(core_axis_name="c", subcore_axis_name="s"),
        compiler_params=pltpu.CompilerParams(needs_layout_passes=False),
        scratch_types=(
            [pltpu.VMEM((CH,), jnp.float32)] * 2
            + [pltpu.VMEM((CH,), jnp.int32)] * 2
            + [pltpu.VMEM((CH,), jnp.float32)] * 2
            + [pltpu.VMEM((CH,), jnp.int32)] * 2
            + [pltpu.VMEM((CH,), jnp.float32)] * 2
            + [pltpu.VMEM((16,), jnp.float32)]
            + [pltpu.SemaphoreType.DMA] * 6
        ),
    )
    def _gather_kernel(xf, pairs, tot, out, *s):
        xv, idxv, fracv, pv, wv = (s[0:2], s[2:4], s[4:6], s[6:8], s[8:10])
        totv = s[10]
        sem_x, sem_g, sem_w = s[11:13], s[13:15], s[15:17]
        _gather_body(xf, pairs, tot, out, xv, idxv, fracv, pv, wv, totv,
                     sem_x, sem_g, sem_w)

    return _gather_kernel


def kernel(x, p):
    pairs, tot = _build_table(p)
    tot16 = jnp.broadcast_to(tot.reshape(1), (16,))
    xf = x.reshape(TOT_X)
    w = _make_gather_kernel()(xf, pairs, tot16)
    return w.reshape(x.shape)


# speedup vs baseline: 2.6039x; 1.0364x over previous
import functools

import jax
import jax.numpy as jnp
from jax import lax
from jax.experimental import pallas as pl
from jax.experimental.pallas import tpu as pltpu
from jax.experimental.pallas import tpu_sc as plsc

N = 1000000
ROWS = N // 128
PAD_N = 1048576
PAD_ROWS = PAD_N // 128
CHUNK_ROWS = 512
N_CHUNKS = PAD_ROWS // CHUNK_ROWS

TOT_X = 16384 * 100
NW = 32
PER_W = TOT_X // NW
CH = 6400
N_ITERS = PER_W // CH


def _max_kernel(p_ref, m_ref):
    i = pl.program_id(0)

    @pl.when(i == 0)
    def _():
        m_ref[0, 0] = jnp.float32(-jnp.inf)

    m_ref[0, 0] = jnp.maximum(m_ref[0, 0], jnp.max(p_ref[...]))


def _table_kernel(m_ref, p_ref, pair_ref, tot_ref, carry_ref):
    i = pl.program_id(0)

    @pl.when(i == 0)
    def _():
        carry_ref[0, 0] = jnp.float32(0.0)

    e = jnp.exp(p_ref[...] - m_ref[0, 0])

    ids = lax.broadcasted_iota(jnp.int32, (128, 128), 0)
    jds = lax.broadcasted_iota(jnp.int32, (128, 128), 1)
    U = (ids <= jds).astype(jnp.float32)

    C = jnp.dot(e, U, preferred_element_type=jnp.float32)
    r_col = C[:, 127:128]

    ids2 = lax.broadcasted_iota(jnp.int32, (CHUNK_ROWS, CHUNK_ROWS), 0)
    jds2 = lax.broadcasted_iota(jnp.int32, (CHUNK_ROWS, CHUNK_ROWS), 1)
    Ls = (jds2 < ids2).astype(jnp.float32)
    offs = jnp.dot(Ls, r_col, preferred_element_type=jnp.float32)

    carry = carry_ref[0, 0]
    ce = C + offs + carry

    a_bits = lax.bitcast_convert_type(ce - e, jnp.int32)
    e_bits = lax.bitcast_convert_type(e, jnp.int32)
    au = (a_bits + 0x8000) >> 16
    eu = (e_bits + 0x8000) >> 16
    pair_ref[...] = (au << 16) | (eu & 0xFFFF)

    new_carry = carry + jnp.sum(r_col)
    carry_ref[0, 0] = new_carry
    tot_ref[0, 0] = new_carry


def _build_table(p):
    p_pad = jnp.full((PAD_N,), -jnp.inf, dtype=jnp.float32).at[:N].set(p)
    p2 = p_pad.reshape(PAD_ROWS, 128)

    m = pl.pallas_call(
        _max_kernel,
        grid=(N_CHUNKS,),
        in_specs=[pl.BlockSpec((CHUNK_ROWS, 128), lambda i: (i, 0))],
        out_specs=pl.BlockSpec(memory_space=pltpu.SMEM),
        out_shape=jax.ShapeDtypeStruct((1, 1), jnp.float32),
    )(p2)

    pairs, tot = pl.pallas_call(
        _table_kernel,
        grid=(N_CHUNKS,),
        in_specs=[
            pl.BlockSpec(memory_space=pltpu.SMEM),
            pl.BlockSpec((CHUNK_ROWS, 128), lambda i: (i, 0)),
        ],
        out_specs=[
            pl.BlockSpec((CHUNK_ROWS, 128), lambda i: (i, 0)),
            pl.BlockSpec(memory_space=pltpu.SMEM),
        ],
        out_shape=[
            jax.ShapeDtypeStruct((PAD_ROWS, 128), jnp.int32),
            jax.ShapeDtypeStruct((1, 1), jnp.float32),
        ],
        scratch_shapes=[pltpu.SMEM((1, 1), jnp.float32)],
    )(m, p2)
    return pairs.reshape(PAD_N), tot


def _gather_body(xf, pairs, tot, out, xv, idxv, fracv, pv, wv, totv,
                 sem_x, sem_g, sem_w):
    wid = lax.axis_index("s") * 2 + lax.axis_index("c")
    pltpu.sync_copy(tot, totv)
    inv = 1.0 / totv[...]

    def base(c):
        return wid * PER_W + c * CH

    def cidx(b):
        @plsc.parallel_loop(0, CH, 16, unroll=8)
        def idx_body(j):
            xs = xv[b][pl.ds(j, 16)]
            t = xs * jnp.float32(N) - jnp.float32(1e-5)
            n = t.astype(jnp.int32)
            idxv[b][pl.ds(j, 16)] = n
            fracv[b][pl.ds(j, 16)] = (
                xs * jnp.float32(N) - n.astype(jnp.float32))

    def cw(b):
        @plsc.parallel_loop(0, CH, 16, unroll=8)
        def w_body(j):
            v = pv[b][pl.ds(j, 16)]
            a = plsc.bitcast(v & jnp.int32(-65536), jnp.float32)
            e = plsc.bitcast(v << 16, jnp.float32)
            g = fracv[b][pl.ds(j, 16)]
            wv[b][pl.ds(j, 16)] = (a + g * e) * inv

    x_cp = [None, None]
    g_cp = [None, None]
    w_cp = [None, None]

    x_cp[0] = pltpu.async_copy(xf.at[pl.ds(base(0), CH)], xv[0], sem_x[0])
    for c in range(N_ITERS):
        b = c & 1
        nb = b ^ 1
        x_cp[b].wait()
        cidx(b)
        if c >= 1:
            g_cp[nb].wait()
        g_cp[b] = pltpu.async_copy(pairs.at[idxv[b]], pv[b], sem_g[b])
        if c + 1 < N_ITERS:
            x_cp[nb] = pltpu.async_copy(
                xf.at[pl.ds(base(c + 1), CH)], xv[nb], sem_x[nb])
        if c >= 1:
            if w_cp[nb] is not None:
                w_cp[nb].wait()
            cw(nb)
            w_cp[nb] = pltpu.async_copy(
                wv[nb], out.at[pl.ds(base(c - 1), CH)], sem_w[nb])
    lb = (N_ITERS - 1) & 1
    g_cp[lb].wait()
    if w_cp[lb] is not None:
        w_cp[lb].wait()
    cw(lb)
    w_cp[lb] = pltpu.async_copy(
        wv[lb], out.at[pl.ds(base(N_ITERS - 1), CH)], sem_w[lb])
    w_cp[lb ^ 1].wait()
    w_cp[lb].wait()


@functools.lru_cache(maxsize=None)
def _make_gather_kernel():
    @functools.partial(
        pl.kernel,
        out_type=jax.ShapeDtypeStruct((TOT_X,), jnp.float32),
        mesh=plsc.VectorSubcoreMesh(core_axis_name="c", subcore_axis_name="s"),
        compiler_params=pltpu.CompilerParams(needs_layout_passes=False),
        scratch_types=(
            [pltpu.VMEM((CH,), jnp.float32)] * 2
            + [pltpu.VMEM((CH,), jnp.int32)] * 2
            + [pltpu.VMEM((CH,), jnp.float32)] * 2
            + [pltpu.VMEM((CH,), jnp.int32)] * 2
            + [pltpu.VMEM((CH,), jnp.float32)] * 2
            + [pltpu.VMEM((16,), jnp.float32)]
            + [pltpu.SemaphoreType.DMA] * 6
        ),
    )
    def _gather_kernel(xf, pairs, tot, out, *s):
        xv, idxv, fracv, pv, wv = (s[0:2], s[2:4], s[4:6], s[6:8], s[8:10])
        totv = s[10]
        sem_x, sem_g, sem_w = s[11:13], s[13:15], s[15:17]
        _gather_body(xf, pairs, tot, out, xv, idxv, fracv, pv, wv, totv,
                     sem_x, sem_g, sem_w)

    return _gather_kernel


def kernel(x, p):
    pairs, tot = _build_table(p)
    tot16 = jnp.broadcast_to(tot.reshape(1), (16,))
    xf = x.reshape(TOT_X)
    w = _make_gather_kernel()(xf, pairs, tot16)
    return w.reshape(x.shape)


# speedup vs baseline: 2.6048x; 1.0003x over previous
import functools

import jax
import jax.numpy as jnp
from jax import lax
from jax.experimental import pallas as pl
from jax.experimental.pallas import tpu as pltpu
from jax.experimental.pallas import tpu_sc as plsc

N = 1000000
ROWS = N // 128
PAD_N = 1048576
PAD_ROWS = PAD_N // 128
CHUNK_ROWS = 512
N_CHUNKS = PAD_ROWS // CHUNK_ROWS

TOT_X = 16384 * 100
NW = 32
PER_W = TOT_X // NW
CH = 6400
N_ITERS = PER_W // CH


def _max_kernel(p_ref, m_ref):
    i = pl.program_id(0)

    @pl.when(i == 0)
    def _():
        m_ref[0, 0] = jnp.float32(-jnp.inf)

    m_ref[0, 0] = jnp.maximum(m_ref[0, 0], jnp.max(p_ref[...]))


def _table_kernel(m_ref, p_ref, pair_ref, tot_ref, carry_ref):
    i = pl.program_id(0)

    @pl.when(i == 0)
    def _():
        carry_ref[0, 0] = jnp.float32(0.0)

    e = jnp.exp(p_ref[...] - m_ref[0, 0])

    ids = lax.broadcasted_iota(jnp.int32, (128, 128), 0)
    jds = lax.broadcasted_iota(jnp.int32, (128, 128), 1)
    U = (ids <= jds).astype(jnp.float32)

    C = jnp.dot(e, U, preferred_element_type=jnp.float32)
    r_col = C[:, 127:128]

    ids2 = lax.broadcasted_iota(jnp.int32, (CHUNK_ROWS, CHUNK_ROWS), 0)
    jds2 = lax.broadcasted_iota(jnp.int32, (CHUNK_ROWS, CHUNK_ROWS), 1)
    Ls = (jds2 < ids2).astype(jnp.float32)
    offs = jnp.dot(Ls, r_col, preferred_element_type=jnp.float32)

    carry = carry_ref[0, 0]
    ce = C + offs + carry

    a_bits = lax.bitcast_convert_type(ce - e, jnp.int32)
    e_bits = lax.bitcast_convert_type(e, jnp.int32)
    au = (a_bits + 0x8000) >> 16
    eu = (e_bits + 0x8000) >> 16
    packed = (au << 16) | (eu & 0xFFFF)
    pair_ref[...] = packed.reshape(CHUNK_ROWS * 128)

    new_carry = carry + jnp.sum(r_col)
    carry_ref[0, 0] = new_carry
    tot_ref[0, 0] = new_carry


def _build_table(p):
    p_pad = jnp.full((PAD_N,), -jnp.inf, dtype=jnp.float32).at[:N].set(p)
    p2 = p_pad.reshape(PAD_ROWS, 128)

    m = pl.pallas_call(
        _max_kernel,
        grid=(N_CHUNKS,),
        in_specs=[pl.BlockSpec((CHUNK_ROWS, 128), lambda i: (i, 0))],
        out_specs=pl.BlockSpec(memory_space=pltpu.SMEM),
        out_shape=jax.ShapeDtypeStruct((1, 1), jnp.float32),
    )(p2)

    pairs, tot = pl.pallas_call(
        _table_kernel,
        grid=(N_CHUNKS,),
        in_specs=[
            pl.BlockSpec(memory_space=pltpu.SMEM),
            pl.BlockSpec((CHUNK_ROWS, 128), lambda i: (i, 0)),
        ],
        out_specs=[
            pl.BlockSpec((CHUNK_ROWS * 128,), lambda i: (i,)),
            pl.BlockSpec(memory_space=pltpu.SMEM),
        ],
        out_shape=[
            jax.ShapeDtypeStruct((PAD_N,), jnp.int32),
            jax.ShapeDtypeStruct((1, 1), jnp.float32),
        ],
        scratch_shapes=[pltpu.SMEM((1, 1), jnp.float32)],
    )(m, p2)
    return pairs, tot


def _gather_body(xf, pairs, tot, out, xv, idxv, fracv, pv, wv, totv,
                 sem_x, sem_g, sem_w):
    wid = lax.axis_index("s") * 2 + lax.axis_index("c")
    pltpu.sync_copy(tot, totv)
    inv = 1.0 / totv[...]

    def base(c):
        return wid * PER_W + c * CH

    def cidx(b):
        @plsc.parallel_loop(0, CH, 16, unroll=8)
        def idx_body(j):
            xs = xv[b][pl.ds(j, 16)]
            t = xs * jnp.float32(N) - jnp.float32(1e-5)
            n = t.astype(jnp.int32)
            idxv[b][pl.ds(j, 16)] = n
            fracv[b][pl.ds(j, 16)] = (
                xs * jnp.float32(N) - n.astype(jnp.float32))

    def cw(b):
        @plsc.parallel_loop(0, CH, 16, unroll=8)
        def w_body(j):
            v = pv[b][pl.ds(j, 16)]
            a = plsc.bitcast(v & jnp.int32(-65536), jnp.float32)
            e = plsc.bitcast(v << 16, jnp.float32)
            g = fracv[b][pl.ds(j, 16)]
            wv[b][pl.ds(j, 16)] = (a + g * e) * inv

    x_cp = [None, None]
    g_cp = [None, None]
    w_cp = [None, None]

    x_cp[0] = pltpu.async_copy(xf.at[pl.ds(base(0), CH)], xv[0], sem_x[0])
    for c in range(N_ITERS):
        b = c & 1
        nb = b ^ 1
        x_cp[b].wait()
        cidx(b)
        if c >= 1:
            g_cp[nb].wait()
        g_cp[b] = pltpu.async_copy(pairs.at[idxv[b]], pv[b], sem_g[b])
        if c + 1 < N_ITERS:
            x_cp[nb] = pltpu.async_copy(
                xf.at[pl.ds(base(c + 1), CH)], xv[nb], sem_x[nb])
        if c >= 1:
            if w_cp[nb] is not None:
                w_cp[nb].wait()
            cw(nb)
            w_cp[nb] = pltpu.async_copy(
                wv[nb], out.at[pl.ds(base(c - 1), CH)], sem_w[nb])
    lb = (N_ITERS - 1) & 1
    g_cp[lb].wait()
    if w_cp[lb] is not None:
        w_cp[lb].wait()
    cw(lb)
    w_cp[lb] = pltpu.async_copy(
        wv[lb], out.at[pl.ds(base(N_ITERS - 1), CH)], sem_w[lb])
    w_cp[lb ^ 1].wait()
    w_cp[lb].wait()


@functools.lru_cache(maxsize=None)
def _make_gather_kernel():
    @functools.partial(
        pl.kernel,
        out_type=jax.ShapeDtypeStruct((TOT_X,), jnp.float32),
        mesh=plsc.VectorSubcoreMesh(core_axis_name="c", subcore_axis_name="s"),
        compiler_params=pltpu.CompilerParams(needs_layout_passes=False),
        scratch_types=(
            [pltpu.VMEM((CH,), jnp.float32)] * 2
            + [pltpu.VMEM((CH,), jnp.int32)] * 2
            + [pltpu.VMEM((CH,), jnp.float32)] * 2
            + [pltpu.VMEM((CH,), jnp.int32)] * 2
            + [pltpu.VMEM((CH,), jnp.float32)] * 2
            + [pltpu.VMEM((16,), jnp.float32)]
            + [pltpu.SemaphoreType.DMA] * 6
        ),
    )
    def _gather_kernel(xf, pairs, tot, out, *s):
        xv, idxv, fracv, pv, wv = (s[0:2], s[2:4], s[4:6], s[6:8], s[8:10])
        totv = s[10]
        sem_x, sem_g, sem_w = s[11:13], s[13:15], s[15:17]
        _gather_body(xf, pairs, tot, out, xv, idxv, fracv, pv, wv, totv,
                     sem_x, sem_g, sem_w)

    return _gather_kernel


def kernel(x, p):
    pairs, tot = _build_table(p)
    tot16 = jnp.broadcast_to(tot.reshape(1), (16,))
    xf = x.reshape(TOT_X)
    w = _make_gather_kernel()(xf, pairs, tot16)
    return w.reshape(x.shape)


# speedup vs baseline: 2.6911x; 1.0331x over previous
import functools

import jax
import jax.numpy as jnp
from jax import lax
from jax.experimental import pallas as pl
from jax.experimental.pallas import tpu as pltpu
from jax.experimental.pallas import tpu_sc as plsc

N = 1000000
ROWS = N // 128
PAD_N = 1048576
PAD_ROWS = PAD_N // 128
CHUNK_ROWS = 512
N_CHUNKS = PAD_ROWS // CHUNK_ROWS

TOT_X = 16384 * 100
NW = 32
PER_W = TOT_X // NW
CH = 6400
N_ITERS = PER_W // CH


def _max_kernel(p_ref, m_ref):
    i = pl.program_id(0)

    @pl.when(i == 0)
    def _():
        m_ref[0, 0] = jnp.float32(-jnp.inf)

    m_ref[0, 0] = jnp.maximum(m_ref[0, 0], jnp.max(p_ref[...]))


def _table_kernel(m_ref, p_ref, pair_ref, tot_ref, carry_ref):
    i = pl.program_id(0)

    @pl.when(i == 0)
    def _():
        carry_ref[0, 0] = jnp.float32(0.0)

    e = jnp.exp(p_ref[...] - m_ref[0, 0])

    ids = lax.broadcasted_iota(jnp.int32, (128, 128), 0)
    jds = lax.broadcasted_iota(jnp.int32, (128, 128), 1)
    U = (ids <= jds).astype(jnp.float32)

    C = jnp.dot(e, U, preferred_element_type=jnp.float32)
    r_col = C[:, 127:128]

    ids2 = lax.broadcasted_iota(jnp.int32, (CHUNK_ROWS, CHUNK_ROWS), 0)
    jds2 = lax.broadcasted_iota(jnp.int32, (CHUNK_ROWS, CHUNK_ROWS), 1)
    Ls = (jds2 < ids2).astype(jnp.float32)
    offs = jnp.dot(Ls, r_col, preferred_element_type=jnp.float32)

    carry = carry_ref[0, 0]
    ce = C + offs + carry

    a_bits = lax.bitcast_convert_type(ce - e, jnp.int32)
    e_bits = lax.bitcast_convert_type(e, jnp.int32)
    au = (a_bits + 0x8000) >> 16
    eu = (e_bits + 0x8000) >> 16
    packed = (au << 16) | (eu & 0xFFFF)
    pair_ref[...] = packed.reshape(CHUNK_ROWS * 128)

    new_carry = carry + jnp.sum(r_col)
    carry_ref[0, 0] = new_carry
    tot_ref[0, 0] = new_carry


def _build_table(p):
    p_pad = jnp.full((PAD_N,), -jnp.inf, dtype=jnp.float32).at[:N].set(p)
    p2 = p_pad.reshape(PAD_ROWS, 128)

    m = pl.pallas_call(
        _max_kernel,
        grid=(PAD_ROWS // 2048,),
        in_specs=[pl.BlockSpec((2048, 128), lambda i: (i, 0))],
        out_specs=pl.BlockSpec(memory_space=pltpu.SMEM),
        out_shape=jax.ShapeDtypeStruct((1, 1), jnp.float32),
    )(p2)

    pairs, tot = pl.pallas_call(
        _table_kernel,
        grid=(N_CHUNKS,),
        in_specs=[
            pl.BlockSpec(memory_space=pltpu.SMEM),
            pl.BlockSpec((CHUNK_ROWS, 128), lambda i: (i, 0)),
        ],
        out_specs=[
            pl.BlockSpec((CHUNK_ROWS * 128,), lambda i: (i,)),
            pl.BlockSpec(memory_space=pltpu.SMEM),
        ],
        out_shape=[
            jax.ShapeDtypeStruct((PAD_N,), jnp.int32),
            jax.ShapeDtypeStruct((1, 1), jnp.float32),
        ],
        scratch_shapes=[pltpu.SMEM((1, 1), jnp.float32)],
    )(m, p2)
    return pairs, tot


def _gather_body(xf, pairs, tot, out, xv, idxv, fracv, pv, wv, totv,
                 sem_x, sem_g, sem_w):
    wid = lax.axis_index("s") * 2 + lax.axis_index("c")
    pltpu.sync_copy(tot, totv)
    inv = 1.0 / totv[...]

    def base(c):
        return wid * PER_W + c * CH

    def cidx(b):
        @plsc.parallel_loop(0, CH, 16, unroll=8)
        def idx_body(j):
            xs = xv[b][pl.ds(j, 16)]
            t = xs * jnp.float32(N) - jnp.float32(1e-5)
            n = t.astype(jnp.int32)
            idxv[b][pl.ds(j, 16)] = n
            fracv[b][pl.ds(j, 16)] = (
                xs * jnp.float32(N) - n.astype(jnp.float32))

    def cw(b):
        @plsc.parallel_loop(0, CH, 16, unroll=8)
        def w_body(j):
            v = pv[b][pl.ds(j, 16)]
            a = plsc.bitcast(v & jnp.int32(-65536), jnp.float32)
            e = plsc.bitcast(v << 16, jnp.float32)
            g = fracv[b][pl.ds(j, 16)]
            wv[b][pl.ds(j, 16)] = (a + g * e) * inv

    x_cp = [None, None]
    g_cp = [None, None]
    w_cp = [None, None]

    x_cp[0] = pltpu.async_copy(xf.at[pl.ds(base(0), CH)], xv[0], sem_x[0])
    for c in range(N_ITERS):
        b = c & 1
        nb = b ^ 1
        x_cp[b].wait()
        cidx(b)
        if c >= 1:
            g_cp[nb].wait()
        g_cp[b] = pltpu.async_copy(pairs.at[idxv[b]], pv[b], sem_g[b])
        if c + 1 < N_ITERS:
            x_cp[nb] = pltpu.async_copy(
                xf.at[pl.ds(base(c + 1), CH)], xv[nb], sem_x[nb])
        if c >= 1:
            if w_cp[nb] is not None:
                w_cp[nb].wait()
            cw(nb)
            w_cp[nb] = pltpu.async_copy(
                wv[nb], out.at[pl.ds(base(c - 1), CH)], sem_w[nb])
    lb = (N_ITERS - 1) & 1
    g_cp[lb].wait()
    if w_cp[lb] is not None:
        w_cp[lb].wait()
    cw(lb)
    w_cp[lb] = pltpu.async_copy(
        wv[lb], out.at[pl.ds(base(N_ITERS - 1), CH)], sem_w[lb])
    w_cp[lb ^ 1].wait()
    w_cp[lb].wait()


@functools.lru_cache(maxsize=None)
def _make_gather_kernel():
    @functools.partial(
        pl.kernel,
        out_type=jax.ShapeDtypeStruct((TOT_X,), jnp.float32),
        mesh=plsc.VectorSubcoreMesh(core_axis_name="c", subcore_axis_name="s"),
        compiler_params=pltpu.CompilerParams(needs_layout_passes=False),
        scratch_types=(
            [pltpu.VMEM((CH,), jnp.float32)] * 2
            + [pltpu.VMEM((CH,), jnp.int32)] * 2
            + [pltpu.VMEM((CH,), jnp.float32)] * 2
            + [pltpu.VMEM((CH,), jnp.int32)] * 2
            + [pltpu.VMEM((CH,), jnp.float32)] * 2
            + [pltpu.VMEM((16,), jnp.float32)]
            + [pltpu.SemaphoreType.DMA] * 6
        ),
    )
    def _gather_kernel(xf, pairs, tot, out, *s):
        xv, idxv, fracv, pv, wv = (s[0:2], s[2:4], s[4:6], s[6:8], s[8:10])
        totv = s[10]
        sem_x, sem_g, sem_w = s[11:13], s[13:15], s[15:17]
        _gather_body(xf, pairs, tot, out, xv, idxv, fracv, pv, wv, totv,
                     sem_x, sem_g, sem_w)

    return _gather_kernel


def kernel(x, p):
    pairs, tot = _build_table(p)
    tot16 = jnp.broadcast_to(tot.reshape(1), (16,))
    xf = x.reshape(TOT_X)
    w = _make_gather_kernel()(xf, pairs, tot16)
    return w.reshape(x.shape)
